# baseline (speedup 1.0000x reference)
import functools

import jax
import jax.numpy as jnp
import numpy as np
from jax.experimental import pallas as pl
from jax.experimental.pallas import tpu as pltpu

_BQ = 512
_BK = 512
_G = 8
_NEG = -1e30
_NT = 80


def _flash_body(nact_ref, it_ref, jt_ref, kmin_ref, smax_ref, cu_ref,
                q_ref, k_ref, v_ref, o_ref,
                s_ref, acc_ref, m_ref, l_ref, *, num_segs, g, bq, bk):
    t = pl.program_id(1)
    d = q_ref.shape[-1]
    it = it_ref[t]
    jt = jt_ref[t]

    @pl.when(t == 0)
    def _init_acc():
        acc_ref[...] = jnp.zeros_like(acc_ref)
        l_ref[...] = jnp.zeros_like(l_ref)

    @pl.when(t < nact_ref[0])
    def _active():
        kmin = kmin_ref[it]
        smax = smax_ref[it]
        first = jt == kmin
        ones_bk = jnp.ones((bk, 8), jnp.bfloat16)

        for gg in range(g):
            s_ref[gg] = jax.lax.dot_general(
                k_ref[gg], q_ref[gg], (((1,), (1,)), ((), ())),
                preferred_element_type=jnp.float32)

        @pl.when(jt * bk < smax)
        def _segmask():
            qpos = it * bq + jax.lax.broadcasted_iota(jnp.int32, (1, bq), 1)
            seg_start = jnp.zeros((1, bq), jnp.int32)
            for b in range(1, num_segs + 1):
                c = cu_ref[b]
                seg_start = jnp.where(qpos >= c, c, seg_start)
            kpos = jt * bk + jax.lax.broadcasted_iota(jnp.int32, (bk, 1), 0)
            sel = kpos >= seg_start
            for gg in range(g):
                s_ref[gg] = jnp.where(sel, s_ref[gg], _NEG)

        neg_row = jnp.full((1, bq), _NEG, jnp.float32)

        def update(gg, s):
            m_prev = jnp.where(first, neg_row, m_ref[gg, 0:1, :])
            l_prev = l_ref[gg, 0:1, :]
            m_cur = jnp.max(s, axis=0, keepdims=True)
            m_new = jnp.maximum(m_prev, m_cur)
            alpha = jnp.exp(m_prev - m_new)
            p = jnp.exp(s - m_new).astype(jnp.bfloat16)
            lsum = jax.lax.dot_general(
                ones_bk, p, (((0,), (0,)), ((), ())),
                preferred_element_type=jnp.float32)
            l_new = l_prev * alpha + lsum[0:1, :]
            pv = jax.lax.dot_general(
                v_ref[gg], p, (((0,), (0,)), ((), ())),
                preferred_element_type=jnp.float32)
            acc_ref[gg] = acc_ref[gg] * alpha + pv
            m_ref[gg, 0:1, :] = m_new
            l_ref[gg, 0:1, :] = l_new

        def upd_diag(gg):
            tri = (jax.lax.broadcasted_iota(jnp.int32, (bk, bq), 1)
                   >= jax.lax.broadcasted_iota(jnp.int32, (bk, bq), 0))
            update(gg, jnp.where(tri, s_ref[gg], _NEG))

        for gg in range(g):
            jax.lax.cond(jt == it,
                         functools.partial(upd_diag, gg),
                         lambda gg=gg: update(gg, s_ref[gg]))

        @pl.when(jt == it)
        def _emit():
            outs = []
            for gg in range(g):
                inv = 1.0 / l_ref[gg, 0:1, :]
                outs.append((acc_ref[gg] * inv).T)
            o_ref[...] = jnp.stack(outs, axis=1)


def kernel(q, k, v, cu_seqlens_q, cu_seqlens_k):
    total, num_heads, d = q.shape
    num_segs = cu_seqlens_q.shape[0] - 1
    scale = 1.0 / np.sqrt(d)
    assert _BQ == _BK and total % _BQ == 0 and num_heads % _G == 0
    num_q = total // _BQ
    num_hg = num_heads // _G

    qs = (q * scale).astype(jnp.bfloat16).transpose(1, 0, 2)
    ks = k.astype(jnp.bfloat16).transpose(1, 0, 2)
    vs = v.astype(jnp.bfloat16).transpose(1, 0, 2)

    i_idx = jnp.arange(num_q, dtype=jnp.int32)
    qblk = i_idx * _BQ
    seg_first = jnp.searchsorted(cu_seqlens_q, qblk, side="right") - 1
    seg_last = jnp.searchsorted(cu_seqlens_q, qblk + (_BQ - 1), side="right") - 1
    kmin_blk = (cu_seqlens_q[seg_first] // _BK).astype(jnp.int32)
    smax_blk = cu_seqlens_q[seg_last].astype(jnp.int32)

    counts = i_idx - kmin_blk + 1
    ends = jnp.cumsum(counts).astype(jnp.int32)
    starts = ends - counts
    nact = ends[-1:]
    t_idx = jnp.arange(_NT, dtype=jnp.int32)
    i_of_t = jnp.clip(jnp.searchsorted(ends, t_idx, side="right"),
                      0, num_q - 1).astype(jnp.int32)
    j_raw = kmin_blk[i_of_t] + (t_idx - starts[i_of_t])
    j_of_t = jnp.where(t_idx < nact[0], j_raw, i_of_t).astype(jnp.int32)

    body = functools.partial(_flash_body, num_segs=num_segs, g=_G,
                             bq=_BQ, bk=_BK)
    grid_spec = pltpu.PrefetchScalarGridSpec(
        num_scalar_prefetch=6,
        grid=(num_hg, _NT),
        in_specs=[
            pl.BlockSpec((_G, _BQ, d),
                         lambda h, t, nact, it, jt, *_: (h, it[t], 0)),
            pl.BlockSpec((_G, _BK, d),
                         lambda h, t, nact, it, jt, *_: (h, jt[t], 0)),
            pl.BlockSpec((_G, _BK, d),
                         lambda h, t, nact, it, jt, *_: (h, jt[t], 0)),
        ],
        out_specs=pl.BlockSpec((_BQ, _G, d),
                               lambda h, t, nact, it, jt, *_: (it[t], h, 0)),
        scratch_shapes=[
            pltpu.VMEM((_G, _BK, _BQ), jnp.float32),
            pltpu.VMEM((_G, d, _BQ), jnp.float32),
            pltpu.VMEM((_G, 8, _BQ), jnp.float32),
            pltpu.VMEM((_G, 8, _BQ), jnp.float32),
        ],
    )
    out = pl.pallas_call(
        body,
        grid_spec=grid_spec,
        out_shape=jax.ShapeDtypeStruct((total, num_heads, d), jnp.float32),
        compiler_params=pltpu.CompilerParams(
            dimension_semantics=("arbitrary", "arbitrary"),
        ),
    )(nact, i_of_t, j_of_t, kmin_blk, smax_blk, cu_seqlens_q, qs, ks, vs)
    return out
